# baseline (speedup 1.0000x reference)
import dataclasses
import functools

import jax
import jax.numpy as jnp
from jax import lax
from jax.experimental import pallas as pl
from jax.experimental.pallas import tpu as pltpu
from jax.experimental.pallas import tpu_sc as plsc

OUT_K = 256
IN_K = 1024
NLANES = 16

_MESH = plsc.VectorSubcoreMesh(core_axis_name="c", subcore_axis_name="s")
_CP = pltpu.CompilerParams()
if "needs_layout_passes" in pltpu.CompilerParams.__dataclass_fields__:
    _CP = dataclasses.replace(_CP, needs_layout_passes=False)


def _norm_key_body(w_ref, out_ref):
    w = w_ref[...]
    s = w * w
    t = s + jnp.roll(s, -4, axis=1)
    t = t + jnp.roll(t, -2, axis=1)
    t = t + jnp.roll(t, -1, axis=1)
    norm = jnp.sqrt(t)
    out_ref[...] = jnp.bitwise_not(lax.bitcast_convert_type(norm, jnp.int32))


def _norm_keys(ww, rows):
    block = 256
    grid = rows // block
    spec = pl.BlockSpec((block, IN_K * 8), lambda i: (i, 0))
    return pl.pallas_call(
        _norm_key_body,
        grid=(grid,),
        in_specs=[spec],
        out_specs=spec,
        out_shape=jax.ShapeDtypeStruct((rows, IN_K * 8), jnp.int32),
    )(ww)


def _sc_topk_gather(keys, posf, wtsf, rows):
    rows_per = rows // 32

    @functools.partial(
        pl.kernel,
        out_type=(
            jax.ShapeDtypeStruct((rows * OUT_K * 3,), jnp.float32),
            jax.ShapeDtypeStruct((rows * OUT_K * 8,), jnp.float32),
        ),
        mesh=_MESH,
        compiler_params=_CP,
        scratch_types=[
            pltpu.VMEM((IN_K * 8,), jnp.int32),
            pltpu.VMEM((IN_K,), jnp.int32),
            pltpu.VMEM((IN_K,), jnp.int32),
            pltpu.VMEM((IN_K,), jnp.int32),
            pltpu.VMEM((IN_K,), jnp.int32),
            pltpu.VMEM((256,), jnp.int32),
            pltpu.VMEM((256,), jnp.int32),
            pltpu.VMEM((IN_K * 8,), jnp.float32),
            pltpu.VMEM((IN_K * 3,), jnp.float32),
            pltpu.VMEM((OUT_K * 8,), jnp.float32),
            pltpu.VMEM((OUT_K * 3,), jnp.float32),
            pltpu.SemaphoreType.DMA,
            pltpu.SemaphoreType.DMA,
            pltpu.SemaphoreType.DMA,
            pltpu.SemaphoreType.DMA,
        ],
    )
    def k(keys_hbm, pos_hbm, wts_hbm, outp_hbm, outw_hbm,
          kwide, key_a, idx_a, key_b, idx_b, hist, offs, wrow, prow, wout,
          pout, sem_k, sem_w, sem_p, sem_o):
        wid = lax.axis_index("c") * 16 + lax.axis_index("s")

        @pl.loop(0, rows_per)
        def _row(r):
            row = wid * rows_per + r
            pltpu.async_copy(keys_hbm.at[row], kwide, sem_k).wait()
            cw = pltpu.async_copy(wts_hbm.at[pl.ds(row * (IN_K * 8), IN_K * 8)],
                                  wrow, sem_w)
            cp = pltpu.async_copy(pos_hbm.at[pl.ds(row * (IN_K * 3), IN_K * 3)],
                                  prow, sem_p)

            for p in range(4):
                src_k, src_i = (key_a, idx_a) if p % 2 == 0 else (key_b, idx_b)
                dst_k, dst_i = (key_b, idx_b) if p % 2 == 0 else (key_a, idx_a)
                shift = 8 * p

                for j in range(16):
                    hist[pl.ds(16 * j, 16)] = jnp.zeros((16,), jnp.int32)

                @pl.loop(0, IN_K, step=NLANES)
                def _hist(c0):
                    if p == 0:
                        kk = plsc.load_gather(
                            kwide, [(lax.iota(jnp.int32, NLANES) + c0) * 8])
                    else:
                        kk = src_k[pl.ds(c0, NLANES)]
                    d = lax.shift_right_logical(kk, shift) & 255
                    cnt, lastm = plsc.scan_count(d)
                    plsc.addupdate_scatter(hist, [d], cnt.astype(jnp.int32),
                                           mask=lastm)

                carry = jnp.int32(0)
                for j in range(16):
                    h = hist[pl.ds(16 * j, 16)]
                    offs[pl.ds(16 * j, 16)] = plsc.cumsum(h) - h + carry
                    carry = carry + jnp.sum(h)

                @pl.loop(0, IN_K, step=NLANES)
                def _perm(c0):
                    if p == 0:
                        kk = plsc.load_gather(
                            kwide, [(lax.iota(jnp.int32, NLANES) + c0) * 8])
                        vv = lax.iota(jnp.int32, NLANES) + c0
                    else:
                        kk = src_k[pl.ds(c0, NLANES)]
                        vv = src_i[pl.ds(c0, NLANES)]
                    d = lax.shift_right_logical(kk, shift) & 255
                    cnt, lastm = plsc.scan_count(d)
                    cnt = cnt.astype(jnp.int32)
                    base = plsc.load_gather(offs, [d])
                    pos = base + cnt - 1
                    plsc.store_scatter(dst_k, [pos], kk)
                    plsc.store_scatter(dst_i, [pos], vv)
                    plsc.addupdate_scatter(offs, [d], cnt, mask=lastm)

            cw.wait()
            cp.wait()

            for i in range(OUT_K // NLANES):
                sel = idx_a[pl.ds(NLANES * i, NLANES)]
                dst = lax.iota(jnp.int32, NLANES) + NLANES * i
                for c in range(8):
                    vals = plsc.load_gather(wrow, [sel * 8 + c])
                    plsc.store_scatter(wout, [dst * 8 + c], vals)
                for c in range(3):
                    vals = plsc.load_gather(prow, [sel * 3 + c])
                    plsc.store_scatter(pout, [dst * 3 + c], vals)

            pltpu.async_copy(
                pout, outp_hbm.at[pl.ds(row * (OUT_K * 3), OUT_K * 3)],
                sem_o).wait()
            pltpu.async_copy(
                wout, outw_hbm.at[pl.ds(row * (OUT_K * 8), OUT_K * 8)],
                sem_o).wait()

    return k(keys, posf, wtsf)


def kernel(positions, weights):
    b, c, in_k, _ = positions.shape
    rows = b * c
    keys = _norm_keys(weights.reshape(rows, in_k * 8), rows)
    posf = positions.reshape(rows * in_k * 3)
    wtsf = weights.reshape(rows * in_k * 8)
    outp, outw = _sc_topk_gather(keys, posf, wtsf, rows)
    return (outp.reshape(b, c, OUT_K, 3), outw.reshape(b, c, OUT_K, 8))


# speedup vs baseline: 8.1904x; 8.1904x over previous
import dataclasses
import functools

import jax
import jax.numpy as jnp
from jax import lax
from jax.experimental import pallas as pl
from jax.experimental.pallas import tpu as pltpu
from jax.experimental.pallas import tpu_sc as plsc

OUT_K = 256
IN_K = 1024
NLANES = 16

_MESH = plsc.VectorSubcoreMesh(core_axis_name="c", subcore_axis_name="s")
_CP = pltpu.CompilerParams()
if "needs_layout_passes" in pltpu.CompilerParams.__dataclass_fields__:
    _CP = dataclasses.replace(_CP, needs_layout_passes=False)


def _norm_key_body(w_ref, out_ref):
    w = w_ref[...]
    s = w * w
    acc = ((s[:, 0, :] + s[:, 4, :]) + (s[:, 2, :] + s[:, 6, :])) + (
        (s[:, 1, :] + s[:, 5, :]) + (s[:, 3, :] + s[:, 7, :]))
    norm = jnp.sqrt(acc)
    out_ref[...] = jnp.bitwise_not(lax.bitcast_convert_type(norm, jnp.int32))


def _norm_keys(wt, rows):
    block = 128
    grid = rows // block
    return pl.pallas_call(
        _norm_key_body,
        grid=(grid,),
        in_specs=[pl.BlockSpec((block, 8, IN_K), lambda i: (i, 0, 0))],
        out_specs=pl.BlockSpec((block, IN_K), lambda i: (i, 0)),
        out_shape=jax.ShapeDtypeStruct((rows, IN_K), jnp.int32),
    )(wt)


def _sc_topk_gather(keys, posf, wtsf, rows):
    rows_per = rows // 32

    @functools.partial(
        pl.kernel,
        out_type=(
            jax.ShapeDtypeStruct((rows * 3 * OUT_K,), jnp.float32),
            jax.ShapeDtypeStruct((rows * 8 * OUT_K,), jnp.float32),
        ),
        mesh=_MESH,
        compiler_params=_CP,
        scratch_types=[
            pltpu.VMEM((IN_K,), jnp.int32),
            pltpu.VMEM((IN_K,), jnp.int32),
            pltpu.VMEM((IN_K,), jnp.int32),
            pltpu.VMEM((IN_K,), jnp.int32),
            pltpu.VMEM((256,), jnp.int32),
            pltpu.VMEM((256,), jnp.int32),
            pltpu.VMEM((8 * IN_K,), jnp.float32),
            pltpu.VMEM((3 * IN_K,), jnp.float32),
            pltpu.VMEM((8 * OUT_K,), jnp.float32),
            pltpu.VMEM((3 * OUT_K,), jnp.float32),
            pltpu.SemaphoreType.DMA,
            pltpu.SemaphoreType.DMA,
            pltpu.SemaphoreType.DMA,
            pltpu.SemaphoreType.DMA,
        ],
    )
    def k(keys_hbm, pos_hbm, wts_hbm, outp_hbm, outw_hbm,
          key_a, idx_a, key_b, idx_b, hist, offs, wrow, prow, wout,
          pout, sem_k, sem_w, sem_p, sem_o):
        wid = lax.axis_index("c") * 16 + lax.axis_index("s")

        @pl.loop(0, rows_per)
        def _row(r):
            row = wid * rows_per + r
            ck = pltpu.async_copy(keys_hbm.at[row], key_a, sem_k)
            cw = pltpu.async_copy(wts_hbm.at[pl.ds(row * (8 * IN_K), 8 * IN_K)],
                                  wrow, sem_w)
            cp = pltpu.async_copy(pos_hbm.at[pl.ds(row * (3 * IN_K), 3 * IN_K)],
                                  prow, sem_p)
            ck.wait()

            for p in range(4):
                src_k, src_i = (key_a, idx_a) if p % 2 == 0 else (key_b, idx_b)
                dst_k, dst_i = (key_b, idx_b) if p % 2 == 0 else (key_a, idx_a)
                shift = 8 * p

                for j in range(16):
                    hist[pl.ds(16 * j, 16)] = jnp.zeros((16,), jnp.int32)

                @pl.loop(0, IN_K, step=4 * NLANES)
                def _hist(c0):
                    for u in range(4):
                        kk = src_k[pl.ds(c0 + u * NLANES, NLANES)]
                        d = lax.shift_right_logical(kk, shift) & 255
                        cnt, lastm = plsc.scan_count(d)
                        plsc.addupdate_scatter(hist, [d],
                                               cnt.astype(jnp.int32),
                                               mask=lastm)

                carry = jnp.int32(0)
                for j in range(16):
                    h = hist[pl.ds(16 * j, 16)]
                    offs[pl.ds(16 * j, 16)] = plsc.cumsum(h) - h + carry
                    carry = carry + jnp.sum(h)

                @pl.loop(0, IN_K, step=NLANES)
                def _perm(c0):
                    kk = src_k[pl.ds(c0, NLANES)]
                    if p == 0:
                        vv = lax.iota(jnp.int32, NLANES) + c0
                    else:
                        vv = src_i[pl.ds(c0, NLANES)]
                    d = lax.shift_right_logical(kk, shift) & 255
                    cnt, lastm = plsc.scan_count(d)
                    cnt = cnt.astype(jnp.int32)
                    base = plsc.load_gather(offs, [d])
                    pos = base + cnt - 1
                    plsc.store_scatter(dst_k, [pos], kk)
                    plsc.store_scatter(dst_i, [pos], vv)
                    plsc.addupdate_scatter(offs, [d], cnt, mask=lastm)

            cw.wait()
            cp.wait()

            for i in range(OUT_K // NLANES):
                sel = idx_a[pl.ds(NLANES * i, NLANES)]
                for c in range(8):
                    wout[pl.ds(c * OUT_K + NLANES * i, NLANES)] = (
                        plsc.load_gather(wrow, [sel + c * IN_K]))
                for c in range(3):
                    pout[pl.ds(c * OUT_K + NLANES * i, NLANES)] = (
                        plsc.load_gather(prow, [sel + c * IN_K]))

            pltpu.async_copy(
                pout, outp_hbm.at[pl.ds(row * (3 * OUT_K), 3 * OUT_K)],
                sem_o).wait()
            pltpu.async_copy(
                wout, outw_hbm.at[pl.ds(row * (8 * OUT_K), 8 * OUT_K)],
                sem_o).wait()

    return k(keys, posf, wtsf)


def kernel(positions, weights):
    b, c, in_k, _ = positions.shape
    rows = b * c
    wt = weights.transpose(0, 1, 3, 2).reshape(rows, 8, in_k)
    keys = _norm_keys(wt, rows)
    posf = positions.transpose(0, 1, 3, 2).reshape(rows * 3 * in_k)
    wtsf = wt.reshape(rows * 8 * in_k)
    outp, outw = _sc_topk_gather(keys, posf, wtsf, rows)
    return (outp.reshape(b, c, 3, OUT_K).transpose(0, 1, 3, 2),
            outw.reshape(b, c, 8, OUT_K).transpose(0, 1, 3, 2))


# speedup vs baseline: 8.3195x; 1.0158x over previous
import dataclasses
import functools

import jax
import jax.numpy as jnp
from jax import lax
from jax.experimental import pallas as pl
from jax.experimental.pallas import tpu as pltpu
from jax.experimental.pallas import tpu_sc as plsc

OUT_K = 256
IN_K = 1024
NLANES = 16

_MESH = plsc.VectorSubcoreMesh(core_axis_name="c", subcore_axis_name="s")
_CP = pltpu.CompilerParams()
if "needs_layout_passes" in pltpu.CompilerParams.__dataclass_fields__:
    _CP = dataclasses.replace(_CP, needs_layout_passes=False)


def _norm_key_body(w_ref, out_ref):
    w = w_ref[...]
    s = w * w
    acc = ((s[:, 0, :] + s[:, 4, :]) + (s[:, 2, :] + s[:, 6, :])) + (
        (s[:, 1, :] + s[:, 5, :]) + (s[:, 3, :] + s[:, 7, :]))
    norm = jnp.sqrt(acc)
    out_ref[...] = jnp.bitwise_not(lax.bitcast_convert_type(norm, jnp.int32))


def _norm_keys(wt, rows):
    block = 128
    grid = rows // block
    return pl.pallas_call(
        _norm_key_body,
        grid=(grid,),
        in_specs=[pl.BlockSpec((block, 8, IN_K), lambda i: (i, 0, 0))],
        out_specs=pl.BlockSpec((block, IN_K), lambda i: (i, 0)),
        out_shape=jax.ShapeDtypeStruct((rows, IN_K), jnp.int32),
    )(wt)


def _sc_topk_gather(keys, posf, wtsf, rows):
    rows_per = rows // 32

    @functools.partial(
        pl.kernel,
        out_type=(
            jax.ShapeDtypeStruct((rows * 3 * OUT_K,), jnp.float32),
            jax.ShapeDtypeStruct((rows * 8 * OUT_K,), jnp.float32),
        ),
        mesh=_MESH,
        compiler_params=_CP,
        scratch_types=[
            pltpu.VMEM((IN_K,), jnp.int32),
            pltpu.VMEM((IN_K,), jnp.int32),
            pltpu.VMEM((IN_K,), jnp.int32),
            pltpu.VMEM((IN_K,), jnp.int32),
            pltpu.VMEM((IN_K,), jnp.int32),
            pltpu.VMEM((4 * 256,), jnp.int32),
            pltpu.VMEM((4 * 256,), jnp.int32),
            pltpu.VMEM((8 * IN_K,), jnp.float32),
            pltpu.VMEM((3 * IN_K,), jnp.float32),
            pltpu.VMEM((8 * OUT_K,), jnp.float32),
            pltpu.VMEM((3 * OUT_K,), jnp.float32),
            pltpu.SemaphoreType.DMA,
            pltpu.SemaphoreType.DMA,
            pltpu.SemaphoreType.DMA,
            pltpu.SemaphoreType.DMA,
        ],
    )
    def k(keys_hbm, pos_hbm, wts_hbm, outp_hbm, outw_hbm,
          kin, key_a, idx_a, key_b, idx_b, hist2d, offs2d, wrow, prow, wout,
          pout, sem_k, sem_w, sem_p, sem_o):
        wid = lax.axis_index("c") * 16 + lax.axis_index("s")
        row0 = wid * rows_per
        pltpu.async_copy(keys_hbm.at[row0], kin, sem_k)

        @pl.loop(0, rows_per)
        def _row(r):
            row = row0 + r
            pltpu.make_async_copy(keys_hbm.at[row], kin, sem_k).wait()
            cw = pltpu.async_copy(wts_hbm.at[pl.ds(row * (8 * IN_K), 8 * IN_K)],
                                  wrow, sem_w)
            cp = pltpu.async_copy(pos_hbm.at[pl.ds(row * (3 * IN_K), 3 * IN_K)],
                                  prow, sem_p)

            for p in range(4):
                if p == 0:
                    src_k, src_i = kin, idx_a
                    dst_k, dst_i = key_a, idx_a
                elif p == 1:
                    src_k, src_i = key_a, idx_a
                    dst_k, dst_i = key_b, idx_b
                elif p == 2:
                    src_k, src_i = key_b, idx_b
                    dst_k, dst_i = key_a, idx_a
                else:
                    src_k, src_i = key_a, idx_a
                    dst_k, dst_i = key_b, idx_b
                shift = 8 * p

                for j in range(64):
                    hist2d[pl.ds(16 * j, 16)] = jnp.zeros((16,), jnp.int32)

                @pl.loop(0, 256, step=NLANES)
                def _hist(c0):
                    for u in range(4):
                        kk = src_k[pl.ds(c0 + u * 256, NLANES)]
                        d = (lax.shift_right_logical(kk, shift) & 255) + (
                            u * 256)
                        cnt, lastm = plsc.scan_count(d)
                        plsc.addupdate_scatter(hist2d, [d],
                                               cnt.astype(jnp.int32),
                                               mask=lastm)

                carry = jnp.int32(0)
                for j in range(16):
                    h0 = hist2d[pl.ds(16 * j, 16)]
                    h1 = hist2d[pl.ds(256 + 16 * j, 16)]
                    h2 = hist2d[pl.ds(512 + 16 * j, 16)]
                    h3 = hist2d[pl.ds(768 + 16 * j, 16)]
                    tot = (h0 + h1) + (h2 + h3)
                    ex = plsc.cumsum(tot) - tot + carry
                    offs2d[pl.ds(16 * j, 16)] = ex
                    ex1 = ex + h0
                    offs2d[pl.ds(256 + 16 * j, 16)] = ex1
                    ex2 = ex1 + h1
                    offs2d[pl.ds(512 + 16 * j, 16)] = ex2
                    offs2d[pl.ds(768 + 16 * j, 16)] = ex2 + h2
                    carry = carry + jnp.sum(tot)

                @pl.loop(0, 256, step=NLANES)
                def _perm(c0):
                    for u in range(4):
                        kk = src_k[pl.ds(c0 + u * 256, NLANES)]
                        if p == 0:
                            vv = lax.iota(jnp.int32, NLANES) + (c0 + u * 256)
                        else:
                            vv = src_i[pl.ds(c0 + u * 256, NLANES)]
                        d = (lax.shift_right_logical(kk, shift) & 255) + (
                            u * 256)
                        cnt, lastm = plsc.scan_count(d)
                        cnt = cnt.astype(jnp.int32)
                        base = plsc.load_gather(offs2d, [d])
                        pos = base + cnt - 1
                        plsc.store_scatter(dst_k, [pos], kk)
                        plsc.store_scatter(dst_i, [pos], vv)
                        plsc.addupdate_scatter(offs2d, [d], cnt, mask=lastm)

                if p == 0:
                    @pl.when(r + 1 < rows_per)
                    def _():
                        pltpu.async_copy(keys_hbm.at[row + 1], kin, sem_k)

            cw.wait()
            cp.wait()

            @pl.when(r > 0)
            def _():
                pltpu.make_async_copy(
                    pout, outp_hbm.at[pl.ds(0, 3 * OUT_K)], sem_o).wait()
                pltpu.make_async_copy(
                    wout, outw_hbm.at[pl.ds(0, 8 * OUT_K)], sem_o).wait()

            for i in range(OUT_K // NLANES):
                sel = idx_b[pl.ds(NLANES * i, NLANES)]
                for c in range(8):
                    wout[pl.ds(c * OUT_K + NLANES * i, NLANES)] = (
                        plsc.load_gather(wrow, [sel + c * IN_K]))
                for c in range(3):
                    pout[pl.ds(c * OUT_K + NLANES * i, NLANES)] = (
                        plsc.load_gather(prow, [sel + c * IN_K]))

            pltpu.async_copy(
                pout, outp_hbm.at[pl.ds(row * (3 * OUT_K), 3 * OUT_K)],
                sem_o)
            pltpu.async_copy(
                wout, outw_hbm.at[pl.ds(row * (8 * OUT_K), 8 * OUT_K)],
                sem_o)

        pltpu.make_async_copy(
            pout, outp_hbm.at[pl.ds(0, 3 * OUT_K)], sem_o).wait()
        pltpu.make_async_copy(
            wout, outw_hbm.at[pl.ds(0, 8 * OUT_K)], sem_o).wait()

    return k(keys, posf, wtsf)


def kernel(positions, weights):
    b, c, in_k, _ = positions.shape
    rows = b * c
    wt = weights.transpose(0, 1, 3, 2).reshape(rows, 8, in_k)
    keys = _norm_keys(wt, rows)
    posf = positions.transpose(0, 1, 3, 2).reshape(rows * 3 * in_k)
    wtsf = wt.reshape(rows * 8 * in_k)
    outp, outw = _sc_topk_gather(keys, posf, wtsf, rows)
    return (outp.reshape(b, c, 3, OUT_K).transpose(0, 1, 3, 2),
            outw.reshape(b, c, 8, OUT_K).transpose(0, 1, 3, 2))


# speedup vs baseline: 12.7360x; 1.5309x over previous
import dataclasses
import functools

import jax
import jax.numpy as jnp
from jax import lax
from jax.experimental import pallas as pl
from jax.experimental.pallas import tpu as pltpu
from jax.experimental.pallas import tpu_sc as plsc

OUT_K = 256
IN_K = 1024
NLANES = 16

_MESH = plsc.VectorSubcoreMesh(core_axis_name="c", subcore_axis_name="s")
_CP = pltpu.CompilerParams()
if "needs_layout_passes" in pltpu.CompilerParams.__dataclass_fields__:
    _CP = dataclasses.replace(_CP, needs_layout_passes=False)


def _norm_key_body(w_ref, out_ref):
    w = w_ref[...]
    s = w * w
    acc = ((s[:, 0, :] + s[:, 4, :]) + (s[:, 2, :] + s[:, 6, :])) + (
        (s[:, 1, :] + s[:, 5, :]) + (s[:, 3, :] + s[:, 7, :]))
    norm = jnp.sqrt(acc)
    out_ref[...] = jnp.bitwise_not(lax.bitcast_convert_type(norm, jnp.int32))


def _norm_keys(wt, rows):
    block = 128
    grid = rows // block
    return pl.pallas_call(
        _norm_key_body,
        grid=(grid,),
        in_specs=[pl.BlockSpec((block, 8, IN_K), lambda i: (i, 0, 0))],
        out_specs=pl.BlockSpec((block, IN_K), lambda i: (i, 0)),
        out_shape=jax.ShapeDtypeStruct((rows, IN_K), jnp.int32),
    )(wt)


def _sc_topk_gather(keys, posf, wtsf, rows):
    rows_per = rows // 32

    @functools.partial(
        pl.kernel,
        out_type=(
            jax.ShapeDtypeStruct((rows * 3 * OUT_K,), jnp.float32),
            jax.ShapeDtypeStruct((rows * 8 * OUT_K,), jnp.float32),
        ),
        mesh=_MESH,
        compiler_params=_CP,
        scratch_types=[
            pltpu.VMEM((IN_K,), jnp.int32),
            pltpu.VMEM((IN_K,), jnp.int32),
            pltpu.VMEM((IN_K,), jnp.int32),
            pltpu.VMEM((IN_K,), jnp.int32),
            pltpu.VMEM((IN_K,), jnp.int32),
            pltpu.VMEM((4 * 256,), jnp.int32),
            pltpu.VMEM((4 * 256,), jnp.int32),
            pltpu.VMEM((IN_K,), jnp.int32),
            pltpu.VMEM((IN_K,), jnp.int32),
            pltpu.VMEM((8 * IN_K,), jnp.float32),
            pltpu.VMEM((3 * IN_K,), jnp.float32),
            pltpu.VMEM((8 * OUT_K,), jnp.float32),
            pltpu.VMEM((3 * OUT_K,), jnp.float32),
            pltpu.SemaphoreType.DMA,
            pltpu.SemaphoreType.DMA,
            pltpu.SemaphoreType.DMA,
            pltpu.SemaphoreType.DMA,
        ],
    )
    def k(keys_hbm, pos_hbm, wts_hbm, outp_hbm, outw_hbm,
          kin, key_a, idx_a, key_b, idx_b, hist2d, offs2d, cnt_buf,
          last_buf, wrow, prow, wout, pout, sem_k, sem_w, sem_p, sem_o):
        wid = lax.axis_index("c") * 16 + lax.axis_index("s")
        row0 = wid * rows_per
        pltpu.async_copy(keys_hbm.at[row0], kin, sem_k)

        @pl.loop(0, rows_per)
        def _row(r):
            row = row0 + r
            pltpu.make_async_copy(keys_hbm.at[row], kin, sem_k).wait()
            cw = pltpu.async_copy(wts_hbm.at[pl.ds(row * (8 * IN_K), 8 * IN_K)],
                                  wrow, sem_w)
            cp = pltpu.async_copy(pos_hbm.at[pl.ds(row * (3 * IN_K), 3 * IN_K)],
                                  prow, sem_p)

            for p in range(4):
                if p == 0:
                    src_k, src_i = kin, idx_a
                    dst_k, dst_i = key_a, idx_a
                elif p == 1:
                    src_k, src_i = key_a, idx_a
                    dst_k, dst_i = key_b, idx_b
                elif p == 2:
                    src_k, src_i = key_b, idx_b
                    dst_k, dst_i = key_a, idx_a
                else:
                    src_k, src_i = key_a, idx_a
                    dst_k, dst_i = key_b, idx_b
                shift = 8 * p

                for j in range(64):
                    hist2d[pl.ds(16 * j, 16)] = jnp.zeros((16,), jnp.int32)

                @plsc.parallel_loop(0, 256, NLANES, unroll=2)
                def _hist(c0):
                    for u in range(4):
                        kk = src_k[pl.ds(c0 + u * 256, NLANES)]
                        d = (lax.shift_right_logical(kk, shift) & 255) + (
                            u * 256)
                        cnt, lastm = plsc.scan_count(d)
                        cnti = cnt.astype(jnp.int32)
                        cnt_buf[pl.ds(c0 + u * 256, NLANES)] = cnti
                        last_buf[pl.ds(c0 + u * 256, NLANES)] = (
                            lastm.astype(jnp.int32))
                        plsc.addupdate_scatter(hist2d, [d], cnti, mask=lastm)

                carry = jnp.int32(0)
                for j in range(16):
                    h0 = hist2d[pl.ds(16 * j, 16)]
                    h1 = hist2d[pl.ds(256 + 16 * j, 16)]
                    h2 = hist2d[pl.ds(512 + 16 * j, 16)]
                    h3 = hist2d[pl.ds(768 + 16 * j, 16)]
                    tot = (h0 + h1) + (h2 + h3)
                    ex = plsc.cumsum(tot) - tot + carry
                    offs2d[pl.ds(16 * j, 16)] = ex
                    ex1 = ex + h0
                    offs2d[pl.ds(256 + 16 * j, 16)] = ex1
                    ex2 = ex1 + h1
                    offs2d[pl.ds(512 + 16 * j, 16)] = ex2
                    offs2d[pl.ds(768 + 16 * j, 16)] = ex2 + h2
                    carry = carry + jnp.sum(tot)

                @pl.loop(0, 256, step=NLANES)
                def _perm(c0):
                    for u in range(4):
                        kk = src_k[pl.ds(c0 + u * 256, NLANES)]
                        if p == 0:
                            vv = lax.iota(jnp.int32, NLANES) + (c0 + u * 256)
                        else:
                            vv = src_i[pl.ds(c0 + u * 256, NLANES)]
                        d = (lax.shift_right_logical(kk, shift) & 255) + (
                            u * 256)
                        cnt = cnt_buf[pl.ds(c0 + u * 256, NLANES)]
                        lastm = last_buf[pl.ds(c0 + u * 256, NLANES)] != 0
                        base = plsc.load_gather(offs2d, [d])
                        pos = base + cnt - 1
                        plsc.store_scatter(dst_k, [pos], kk)
                        plsc.store_scatter(dst_i, [pos], vv)
                        plsc.addupdate_scatter(offs2d, [d], cnt, mask=lastm)

                if p == 0:
                    @pl.when(r + 1 < rows_per)
                    def _():
                        pltpu.async_copy(keys_hbm.at[row + 1], kin, sem_k)

            cw.wait()
            cp.wait()

            @pl.when(r > 0)
            def _():
                pltpu.make_async_copy(
                    pout, outp_hbm.at[pl.ds(0, 3 * OUT_K)], sem_o).wait()
                pltpu.make_async_copy(
                    wout, outw_hbm.at[pl.ds(0, 8 * OUT_K)], sem_o).wait()

            @plsc.parallel_loop(0, OUT_K, NLANES, unroll=2)
            def _gather(c0):
                sel = idx_b[pl.ds(c0, NLANES)]
                for c in range(8):
                    wout[pl.ds(c * OUT_K + c0, NLANES)] = (
                        plsc.load_gather(wrow, [sel + c * IN_K]))
                for c in range(3):
                    pout[pl.ds(c * OUT_K + c0, NLANES)] = (
                        plsc.load_gather(prow, [sel + c * IN_K]))

            pltpu.async_copy(
                pout, outp_hbm.at[pl.ds(row * (3 * OUT_K), 3 * OUT_K)],
                sem_o)
            pltpu.async_copy(
                wout, outw_hbm.at[pl.ds(row * (8 * OUT_K), 8 * OUT_K)],
                sem_o)

        pltpu.make_async_copy(
            pout, outp_hbm.at[pl.ds(0, 3 * OUT_K)], sem_o).wait()
        pltpu.make_async_copy(
            wout, outw_hbm.at[pl.ds(0, 8 * OUT_K)], sem_o).wait()

    return k(keys, posf, wtsf)


def kernel(positions, weights):
    b, c, in_k, _ = positions.shape
    rows = b * c
    wt = weights.transpose(0, 1, 3, 2).reshape(rows, 8, in_k)
    keys = _norm_keys(wt, rows)
    posf = positions.transpose(0, 1, 3, 2).reshape(rows * 3 * in_k)
    wtsf = wt.reshape(rows * 8 * in_k)
    outp, outw = _sc_topk_gather(keys, posf, wtsf, rows)
    return (outp.reshape(b, c, 3, OUT_K).transpose(0, 1, 3, 2),
            outw.reshape(b, c, 8, OUT_K).transpose(0, 1, 3, 2))
